# baseline (speedup 1.0000x reference)
import functools
import math

import jax
import jax.numpy as jnp
from jax import lax
from jax.experimental import pallas as pl
from jax.experimental.pallas import tpu as pltpu
from jax.experimental.pallas import tpu_sc as plsc

EMBEDDING_DIM = 128
MAX_LEN = 8192
BATCH = 16384
HALF = EMBEDDING_DIM // 2

_info = plsc.get_sparse_core_info()
_NC, _NS = _info.num_cores, _info.num_subcores
_NW = _NC * _NS
_BPW = BATCH // _NW
_GROWS = 272
_CROWS = _BPW - _GROWS


def _pe_table() -> jnp.ndarray:
    position = jnp.arange(MAX_LEN, dtype=jnp.float32).reshape(-1, 1)
    div_term = jnp.exp(
        jnp.arange(0, EMBEDDING_DIM, 2, dtype=jnp.float32)
        * (-math.log(10000.0) / EMBEDDING_DIM)
    )
    ang = position * div_term
    return jnp.stack([jnp.sin(ang), jnp.cos(ang)], axis=-1).reshape(
        MAX_LEN, EMBEDDING_DIM
    )


def _factor_tables():
    div = jnp.exp(
        jnp.arange(0, EMBEDDING_DIM, 2, dtype=jnp.float32)
        * (-math.log(10000.0) / EMBEDDING_DIM)
    )
    coarse = (jnp.arange(128, dtype=jnp.float32) * 64.0)[:, None] * div
    fine = jnp.arange(64, dtype=jnp.float32)[:, None] * div
    return (
        jnp.sin(coarse).reshape(-1),
        jnp.cos(coarse).reshape(-1),
        jnp.sin(fine).reshape(-1),
        jnp.cos(fine).reshape(-1),
    )


@functools.partial(
    pl.kernel,
    mesh=plsc.VectorSubcoreMesh(core_axis_name="c", subcore_axis_name="s"),
    out_type=jax.ShapeDtypeStruct((BATCH, EMBEDDING_DIM), jnp.float32),
    compiler_params=pltpu.CompilerParams(needs_layout_passes=False),
    scratch_types=[
        pltpu.VMEM((_BPW,), jnp.int32),
        pltpu.VMEM((128 * HALF,), jnp.float32),
        pltpu.VMEM((128 * HALF,), jnp.float32),
        pltpu.VMEM((64 * HALF,), jnp.float32),
        pltpu.VMEM((64 * HALF,), jnp.float32),
        pltpu.VMEM((_BPW, EMBEDDING_DIM), jnp.float32),
        pltpu.SemaphoreType.DMA,
    ],
)
def _pe_lookup(table_hbm, sa_hbm, ca_hbm, sb_hbm, cb_hbm, idx_hbm, out_hbm,
               idx_v, sa_v, ca_v, sb_v, cb_v, out_v, sem):
    wid = lax.axis_index("s") * _NC + lax.axis_index("c")
    base = wid * _BPW
    pltpu.sync_copy(idx_hbm.at[pl.ds(base, _BPW)], idx_v)
    gather = pltpu.async_copy(
        table_hbm.at[idx_v.at[pl.ds(0, _GROWS)]],
        out_v.at[pl.ds(0, _GROWS)],
        sem,
    )
    pltpu.sync_copy(sa_hbm, sa_v)
    pltpu.sync_copy(ca_hbm, ca_v)
    pltpu.sync_copy(sb_hbm, sb_v)
    pltpu.sync_copy(cb_hbm, cb_v)

    iota2 = lax.iota(jnp.int32, 16) * 2

    def group(g, carry):
        tv = idx_v[pl.ds(_GROWS + g * 16, 16)]
        for lane in range(16):
            t = tv[lane]
            aoff = (t >> 6) * HALF
            boff = (t & 63) * HALF
            rowv = jnp.broadcast_to(_GROWS + g * 16 + lane, (16,))
            for k0 in range(0, HALF, 16):
                sa = sa_v[pl.ds(aoff + k0, 16)]
                ca = ca_v[pl.ds(aoff + k0, 16)]
                sb = sb_v[pl.ds(boff + k0, 16)]
                cb = cb_v[pl.ds(boff + k0, 16)]
                outs = sa * cb + ca * sb
                outc = ca * cb - sa * sb
                pos = iota2 + 2 * k0
                plsc.store_scatter(out_v, [rowv, pos], outs)
                plsc.store_scatter(out_v, [rowv, pos + 1], outc)
        return carry

    lax.fori_loop(0, _CROWS // 16, group, 0)
    gather.wait()
    pltpu.sync_copy(out_v, out_hbm.at[pl.ds(base, _BPW)])


def kernel(timesteps):
    table = _pe_table()
    sa, ca, sb, cb = _factor_tables()
    return _pe_lookup(table, sa, ca, sb, cb, timesteps.astype(jnp.int32))


# speedup vs baseline: 1.0134x; 1.0134x over previous
import functools
import math

import jax
import jax.numpy as jnp
from jax import lax
from jax.experimental import pallas as pl
from jax.experimental.pallas import tpu as pltpu
from jax.experimental.pallas import tpu_sc as plsc

EMBEDDING_DIM = 128
MAX_LEN = 8192
BATCH = 16384
HALF = EMBEDDING_DIM // 2

_info = plsc.get_sparse_core_info()
_NC, _NS = _info.num_cores, _info.num_subcores
_NW = _NC * _NS
_BPW = BATCH // _NW
_GROWS = 272
_CROWS = _BPW - _GROWS


def _pe_table() -> jnp.ndarray:
    position = jnp.arange(MAX_LEN, dtype=jnp.float32).reshape(-1, 1)
    div_term = jnp.exp(
        jnp.arange(0, EMBEDDING_DIM, 2, dtype=jnp.float32)
        * (-math.log(10000.0) / EMBEDDING_DIM)
    )
    ang = position * div_term
    return jnp.stack([jnp.sin(ang), jnp.cos(ang)], axis=-1).reshape(
        MAX_LEN, EMBEDDING_DIM
    )


def _factor_tables():
    div = jnp.exp(
        jnp.arange(0, EMBEDDING_DIM, 2, dtype=jnp.float32)
        * (-math.log(10000.0) / EMBEDDING_DIM)
    )
    coarse = (jnp.arange(128, dtype=jnp.float32) * 64.0)[:, None] * div
    fine = jnp.arange(64, dtype=jnp.float32)[:, None] * div
    return (
        jnp.sin(coarse).reshape(-1),
        jnp.cos(coarse).reshape(-1),
        jnp.sin(fine).reshape(-1),
        jnp.cos(fine).reshape(-1),
    )


@functools.partial(
    pl.kernel,
    mesh=plsc.VectorSubcoreMesh(core_axis_name="c", subcore_axis_name="s"),
    out_type=jax.ShapeDtypeStruct((BATCH, EMBEDDING_DIM), jnp.float32),
    compiler_params=pltpu.CompilerParams(needs_layout_passes=False),
    scratch_types=[
        pltpu.VMEM((_BPW,), jnp.int32),
        pltpu.VMEM((128 * HALF,), jnp.float32),
        pltpu.VMEM((128 * HALF,), jnp.float32),
        pltpu.VMEM((64 * HALF,), jnp.float32),
        pltpu.VMEM((64 * HALF,), jnp.float32),
        pltpu.VMEM((_GROWS, EMBEDDING_DIM), jnp.float32),
        pltpu.VMEM((_CROWS, EMBEDDING_DIM), jnp.float32),
        pltpu.SemaphoreType.DMA,
    ],
)
def _pe_lookup(table_hbm, sa_hbm, ca_hbm, sb_hbm, cb_hbm, idx_hbm, out_hbm,
               idx_v, sa_v, ca_v, sb_v, cb_v, gbuf, cbuf, sem):
    wid = lax.axis_index("s") * _NC + lax.axis_index("c")
    base = wid * _BPW
    pltpu.sync_copy(idx_hbm.at[pl.ds(base, _BPW)], idx_v)
    pltpu.sync_copy(sa_hbm, sa_v)
    pltpu.sync_copy(ca_hbm, ca_v)
    pltpu.sync_copy(sb_hbm, sb_v)
    pltpu.sync_copy(cb_hbm, cb_v)
    gather = pltpu.async_copy(
        table_hbm.at[idx_v.at[pl.ds(0, _GROWS)]],
        gbuf,
        sem,
    )

    iota2 = lax.iota(jnp.int32, 16) * 2

    def group(g, carry):
        tv = idx_v[pl.ds(_GROWS + g * 16, 16)]
        for lane in range(16):
            t = tv[lane]
            aoff = (t >> 6) * HALF
            boff = (t & 63) * HALF
            rowv = jnp.broadcast_to(g * 16 + lane, (16,))
            for k0 in range(0, HALF, 16):
                sa = sa_v[pl.ds(aoff + k0, 16)]
                ca = ca_v[pl.ds(aoff + k0, 16)]
                sb = sb_v[pl.ds(boff + k0, 16)]
                cb = cb_v[pl.ds(boff + k0, 16)]
                outs = sa * cb + ca * sb
                outc = ca * cb - sa * sb
                pos = iota2 + 2 * k0
                plsc.store_scatter(cbuf, [rowv, pos], outs)
                plsc.store_scatter(cbuf, [rowv, pos + 1], outc)
        return carry

    lax.fori_loop(0, _CROWS // 16, group, 0)
    gather.wait()
    pltpu.sync_copy(gbuf, out_hbm.at[pl.ds(base, _GROWS)])
    pltpu.sync_copy(cbuf, out_hbm.at[pl.ds(base + _GROWS, _CROWS)])


def kernel(timesteps):
    table = _pe_table()
    sa, ca, sb, cb = _factor_tables()
    return _pe_lookup(table, sa, ca, sb, cb, timesteps.astype(jnp.int32))


# speedup vs baseline: 1.3470x; 1.3292x over previous
import functools
import math

import jax
import jax.numpy as jnp
from jax import lax
from jax.experimental import pallas as pl
from jax.experimental.pallas import tpu as pltpu
from jax.experimental.pallas import tpu_sc as plsc

EMBEDDING_DIM = 128
MAX_LEN = 8192
BATCH = 16384

_info = plsc.get_sparse_core_info()
_NC, _NS = _info.num_cores, _info.num_subcores
_NW = _NC * _NS
_BPW = BATCH // _NW


def _pe_table() -> jnp.ndarray:
    position = jnp.arange(MAX_LEN, dtype=jnp.float32).reshape(-1, 1)
    div_term = jnp.exp(
        jnp.arange(0, EMBEDDING_DIM, 2, dtype=jnp.float32)
        * (-math.log(10000.0) / EMBEDDING_DIM)
    )
    ang = position * div_term
    return jnp.stack([jnp.sin(ang), jnp.cos(ang)], axis=-1).reshape(
        MAX_LEN, EMBEDDING_DIM
    )


@functools.partial(
    pl.kernel,
    mesh=plsc.VectorSubcoreMesh(core_axis_name="c", subcore_axis_name="s"),
    out_type=jax.ShapeDtypeStruct((BATCH, EMBEDDING_DIM), jnp.float32),
    compiler_params=pltpu.CompilerParams(
        disable_bounds_checks=True,
        disable_semaphore_checks=True,
        skip_device_barrier=True,
    ),
    scratch_types=[
        pltpu.VMEM((_BPW,), jnp.int32),
        pltpu.VMEM((_BPW, EMBEDDING_DIM), jnp.float32),
        pltpu.SemaphoreType.DMA,
    ],
)
def _gather(table_hbm, idx_hbm, out_hbm, idx_v, rows_v, sem):
    wid = lax.axis_index("s") * _NC + lax.axis_index("c")
    base = wid * _BPW
    pltpu.sync_copy(idx_hbm.at[pl.ds(base, _BPW)], idx_v)
    pltpu.async_copy(table_hbm.at[idx_v], rows_v, sem).wait()
    pltpu.sync_copy(rows_v, out_hbm.at[pl.ds(base, _BPW)])


def kernel(timesteps):
    table = _pe_table()
    return _gather(table, timesteps.astype(jnp.int32))
